# baseline (speedup 1.0000x reference)
import functools

import jax
import jax.numpy as jnp
from jax import lax
from jax.experimental import pallas as pl
from jax.experimental.pallas import tpu as pltpu
from jax.experimental.pallas import tpu_sc as plsc

NUM_EMB = 1000
EMB_DIM = 1024
BATCH = 16384

_info = plsc.get_sparse_core_info()
NC, NS = _info.num_cores, _info.num_subcores
NW = NC * NS
B_PER_W = BATCH // NW
CHUNK = 16
NCH = B_PER_W // CHUNK
NBUF = 6


def _gather_body(idx_hbm, table_hbm, out_hbm, idx_v, rows_v,
                 g0, g1, g2, g3, g4, g5, w0, w1, w2, w3, w4, w5):
    gsems = (g0, g1, g2, g3, g4, g5)
    wsems = (w0, w1, w2, w3, w4, w5)
    wid = lax.axis_index("s") * NC + lax.axis_index("c")
    base = wid * B_PER_W
    pltpu.sync_copy(idx_hbm.at[pl.ds(base, B_PER_W)], idx_v)

    gh = [None] * NBUF
    wh = [None] * NBUF
    for ch in range(NBUF):
        b = ch % NBUF
        gh[b] = pltpu.async_copy(
            table_hbm.at[idx_v.at[pl.ds(ch * CHUNK, CHUNK)]],
            rows_v.at[b], gsems[b])
    for ch in range(NCH):
        b = ch % NBUF
        gh[b].wait()
        wh[b] = pltpu.async_copy(rows_v.at[b],
                                 out_hbm.at[pl.ds(base + ch * CHUNK, CHUNK)],
                                 wsems[b])
        prev = ch - 1
        if prev >= 0 and prev + NBUF < NCH:
            bp = prev % NBUF
            wh[bp].wait()
            gh[bp] = pltpu.async_copy(
                table_hbm.at[idx_v.at[pl.ds((prev + NBUF) * CHUNK, CHUNK)]],
                rows_v.at[bp], gsems[bp])
    for ch in range(NCH - NBUF, NCH):
        if ch >= 0:
            wh[ch % NBUF].wait()


_gather = functools.partial(
    pl.kernel,
    mesh=plsc.VectorSubcoreMesh(core_axis_name="c", subcore_axis_name="s"),
    out_type=jax.ShapeDtypeStruct((BATCH, EMB_DIM), jnp.float32),
    scratch_types=[
        pltpu.VMEM((B_PER_W,), jnp.int32),
        pltpu.VMEM((NBUF, CHUNK, EMB_DIM), jnp.float32),
        pltpu.SemaphoreType.DMA,
        pltpu.SemaphoreType.DMA,
        pltpu.SemaphoreType.DMA,
        pltpu.SemaphoreType.DMA,
        pltpu.SemaphoreType.DMA,
        pltpu.SemaphoreType.DMA,
        pltpu.SemaphoreType.DMA,
        pltpu.SemaphoreType.DMA,
        pltpu.SemaphoreType.DMA,
        pltpu.SemaphoreType.DMA,
        pltpu.SemaphoreType.DMA,
        pltpu.SemaphoreType.DMA,
    ],
)(_gather_body)


@jax.jit
def kernel(x, pos_encoding):
    return _gather(x, pos_encoding)


# speedup vs baseline: 1.0044x; 1.0044x over previous
import functools

import jax
import jax.numpy as jnp
from jax import lax
from jax.experimental import pallas as pl
from jax.experimental.pallas import tpu as pltpu
from jax.experimental.pallas import tpu_sc as plsc

NUM_EMB = 1000
EMB_DIM = 1024
BATCH = 16384

_info = plsc.get_sparse_core_info()
NC, NS = _info.num_cores, _info.num_subcores
NW = NC * NS
B_PER_W = BATCH // NW
CHUNK = 16
NCH = B_PER_W // CHUNK
NBUF = 6


def _gather_body(idx_hbm, table_hbm, out_hbm, idx_v, rows_v,
                 g0, g1, g2, g3, g4, g5, w0, w1, w2, w3, w4, w5):
    gsems = (g0, g1, g2, g3, g4, g5)
    wsems = (w0, w1, w2, w3, w4, w5)
    wid = lax.axis_index("s") * NC + lax.axis_index("c")
    base = wid * B_PER_W
    pltpu.sync_copy(idx_hbm.at[pl.ds(base, B_PER_W)], idx_v)

    gh = [None] * NBUF
    wh = [None] * NBUF
    for ch in range(NBUF):
        b = ch % NBUF
        gh[b] = pltpu.async_copy(
            table_hbm.at[idx_v.at[pl.ds(ch * CHUNK, CHUNK)]],
            rows_v.at[b], gsems[b])
    for ch in range(NCH):
        b = ch % NBUF
        prev = ch - 1
        if prev >= 0 and prev + NBUF < NCH:
            bp = prev % NBUF
            wh[bp].wait()
            gh[bp] = pltpu.async_copy(
                table_hbm.at[idx_v.at[pl.ds((prev + NBUF) * CHUNK, CHUNK)]],
                rows_v.at[bp], gsems[bp])
        gh[b].wait()
        wh[b] = pltpu.async_copy(rows_v.at[b],
                                 out_hbm.at[pl.ds(base + ch * CHUNK, CHUNK)],
                                 wsems[b])
    for ch in range(NCH - NBUF, NCH):
        if ch >= 0:
            wh[ch % NBUF].wait()


_gather = functools.partial(
    pl.kernel,
    mesh=plsc.VectorSubcoreMesh(core_axis_name="c", subcore_axis_name="s"),
    out_type=jax.ShapeDtypeStruct((BATCH, EMB_DIM), jnp.float32),
    scratch_types=[
        pltpu.VMEM((B_PER_W,), jnp.int32),
        pltpu.VMEM((NBUF, CHUNK, EMB_DIM), jnp.float32),
        pltpu.SemaphoreType.DMA,
        pltpu.SemaphoreType.DMA,
        pltpu.SemaphoreType.DMA,
        pltpu.SemaphoreType.DMA,
        pltpu.SemaphoreType.DMA,
        pltpu.SemaphoreType.DMA,
        pltpu.SemaphoreType.DMA,
        pltpu.SemaphoreType.DMA,
        pltpu.SemaphoreType.DMA,
        pltpu.SemaphoreType.DMA,
        pltpu.SemaphoreType.DMA,
        pltpu.SemaphoreType.DMA,
    ],
)(_gather_body)


@jax.jit
def kernel(x, pos_encoding):
    return _gather(x, pos_encoding)
